# Initial kernel scaffold; baseline (speedup 1.0000x reference)
import jax
import jax.numpy as jnp
from jax.experimental import pallas as pl


def kernel(x, emb0, emb1, emb2, emb3, emb4, emb5, emb6, emb7, emb8):
    raise NotImplementedError("write your pallas kernel here")


# speedup vs baseline: 1.1259x; 1.1259x over previous
import functools

import jax
import jax.numpy as jnp
from jax import lax
from jax.experimental import pallas as pl
from jax.experimental.pallas import tpu as pltpu
from jax.experimental.pallas import tpu_sc as plsc

N = 100000
OUT_D = 128
BLK = 400
SUB = 80
NSUB = BLK // SUB
NBLK = N // BLK
NW = 32
ITERS = -(-NBLK // NW)

_PAIRS = ((1, 2, 11), (3, 4, 9), (5, 6, 8), (7, 8, 2))
_OFFS = (0, 64, 80, 96, 112)


def _sc_body(xt_h, emb0_h, p12_h, p34_h, p56_h, p78_h, out_h,
             idx_v, pi_v, buf0_v, bufp12_v, bufp34_v, bufp56_v, bufp78_v,
             gsem):
    wid = lax.axis_index("s") * 2 + lax.axis_index("c")

    def block_body(it, carry):
        blk = it * NW + wid

        @pl.when(blk < NBLK)
        def _():
            base = blk * BLK
            for i in range(9):
                pltpu.sync_copy(xt_h.at[i, pl.ds(base, BLK)], idx_v.at[i])
            for t in range(BLK // 16):
                s = pl.ds(t * 16, 16)
                for k, (a, b, w) in enumerate(_PAIRS):
                    pi_v[k, s] = idx_v[a, s] * w + idx_v[b, s]
            cps = []
            for s in range(NSUB):
                rows = pl.ds(s * SUB, SUB)
                cps.append(pltpu.async_copy(
                    emb0_h.at[idx_v.at[0, rows]], buf0_v.at[rows], gsem))
                for k, (tab, buf) in enumerate((
                        (p12_h, bufp12_v), (p34_h, bufp34_v),
                        (p56_h, bufp56_v), (p78_h, bufp78_v))):
                    cps.append(pltpu.async_copy(
                        tab.at[pi_v.at[k, rows]], buf.at[rows], gsem))
            for cp in cps:
                cp.wait()
            rows_out = pl.ds(base, BLK)
            pltpu.sync_copy(buf0_v, out_h.at[rows_out, pl.ds(0, 64)])
            pltpu.sync_copy(bufp12_v, out_h.at[rows_out, pl.ds(64, 16)])
            pltpu.sync_copy(bufp34_v, out_h.at[rows_out, pl.ds(80, 16)])
            pltpu.sync_copy(bufp56_v, out_h.at[rows_out, pl.ds(96, 16)])
            pltpu.sync_copy(bufp78_v, out_h.at[rows_out, pl.ds(112, 16)])

        return carry

    lax.fori_loop(0, ITERS, block_body, 0)


@functools.partial(jax.jit, static_argnums=())
def _sc_call(xt, emb0, p12, p34, p56, p78):
    mesh = plsc.VectorSubcoreMesh(core_axis_name="c", subcore_axis_name="s")
    fn = pl.kernel(
        _sc_body,
        out_type=jax.ShapeDtypeStruct((N, OUT_D), jnp.float32),
        mesh=mesh,
        compiler_params=pltpu.CompilerParams(use_tc_tiling_on_sc=False),
        scratch_types=[
            pltpu.VMEM((9, BLK), jnp.int32),
            pltpu.VMEM((4, BLK), jnp.int32),
            pltpu.VMEM((BLK, 64), jnp.float32),
            pltpu.VMEM((BLK, 16), jnp.float32),
            pltpu.VMEM((BLK, 16), jnp.float32),
            pltpu.VMEM((BLK, 16), jnp.float32),
            pltpu.VMEM((BLK, 16), jnp.float32),
            pltpu.SemaphoreType.DMA,
        ],
    )
    return fn(xt, emb0, p12, p34, p56, p78)


def _pair(a, b):
    na, da = a.shape
    nb, db = b.shape
    left = jnp.broadcast_to(a[:, None, :], (na, nb, da))
    right = jnp.broadcast_to(b[None, :, :], (na, nb, db))
    return jnp.concatenate([left, right], axis=-1).reshape(na * nb, da + db)


def kernel(x, emb0, emb1, emb2, emb3, emb4, emb5, emb6, emb7, emb8):
    xt = x.T
    p12 = _pair(emb1, emb2)
    p34 = _pair(emb3, emb4)
    p56 = _pair(emb5, emb6)
    p78 = _pair(emb7, emb8)
    return _sc_call(xt, emb0, p12, p34, p56, p78)


# speedup vs baseline: 1.1272x; 1.0012x over previous
import functools

import jax
import jax.numpy as jnp
from jax import lax
from jax.experimental import pallas as pl
from jax.experimental.pallas import tpu as pltpu
from jax.experimental.pallas import tpu_sc as plsc

N = 100000
OUT_D = 128
BLK = 400
SUB = 80
NSUB = BLK // SUB
NBLK = N // BLK
NW = 32
ITERS = -(-NBLK // NW)

_PAIRS = ((1, 2, 11), (3, 4, 9), (5, 6, 8), (7, 8, 2))
_OFFS = (0, 64, 80, 96, 112)


def _sc_body(xt_h, emb0_h, p12_h, p34_h, p56_h, p78_h, out_h,
             idx_v, pi_v, buf0_v, bufp12_v, bufp34_v, bufp56_v, bufp78_v,
             gsem, wsem):
    wid = lax.axis_index("s") * 2 + lax.axis_index("c")

    def block_body(it, carry):
        blk = it * NW + wid

        @pl.when(blk < NBLK)
        def _():
            base = blk * BLK
            pltpu.sync_copy(xt_h.at[:, pl.ds(base, BLK)], idx_v)
            for t in range(BLK // 16):
                s = pl.ds(t * 16, 16)
                for k, (a, b, w) in enumerate(_PAIRS):
                    pi_v[k, s] = idx_v[a, s] * w + idx_v[b, s]
            cps = [pltpu.async_copy(
                emb0_h.at[idx_v.at[0]], buf0_v, gsem)]
            for k, (tab, buf) in enumerate((
                    (p12_h, bufp12_v), (p34_h, bufp34_v),
                    (p56_h, bufp56_v), (p78_h, bufp78_v))):
                cps.append(pltpu.async_copy(tab.at[pi_v.at[k]], buf, gsem))
            for cp in cps:
                cp.wait()
            rows_out = pl.ds(base, BLK)
            wps = [
                pltpu.async_copy(buf0_v, out_h.at[rows_out, pl.ds(0, 64)], wsem),
                pltpu.async_copy(bufp12_v, out_h.at[rows_out, pl.ds(64, 16)], wsem),
                pltpu.async_copy(bufp34_v, out_h.at[rows_out, pl.ds(80, 16)], wsem),
                pltpu.async_copy(bufp56_v, out_h.at[rows_out, pl.ds(96, 16)], wsem),
                pltpu.async_copy(bufp78_v, out_h.at[rows_out, pl.ds(112, 16)], wsem),
            ]
            for wp in wps:
                wp.wait()

        return carry

    lax.fori_loop(0, ITERS, block_body, 0)


@functools.partial(jax.jit, static_argnums=())
def _sc_call(xt, emb0, p12, p34, p56, p78):
    mesh = plsc.VectorSubcoreMesh(core_axis_name="c", subcore_axis_name="s")
    fn = pl.kernel(
        _sc_body,
        out_type=jax.ShapeDtypeStruct((N, OUT_D), jnp.float32),
        mesh=mesh,
        compiler_params=pltpu.CompilerParams(use_tc_tiling_on_sc=False),
        scratch_types=[
            pltpu.VMEM((9, BLK), jnp.int32),
            pltpu.VMEM((4, BLK), jnp.int32),
            pltpu.VMEM((BLK, 64), jnp.float32),
            pltpu.VMEM((BLK, 16), jnp.float32),
            pltpu.VMEM((BLK, 16), jnp.float32),
            pltpu.VMEM((BLK, 16), jnp.float32),
            pltpu.VMEM((BLK, 16), jnp.float32),
            pltpu.SemaphoreType.DMA,
            pltpu.SemaphoreType.DMA,
        ],
    )
    return fn(xt, emb0, p12, p34, p56, p78)


def _pair(a, b):
    na, da = a.shape
    nb, db = b.shape
    left = jnp.broadcast_to(a[:, None, :], (na, nb, da))
    right = jnp.broadcast_to(b[None, :, :], (na, nb, db))
    return jnp.concatenate([left, right], axis=-1).reshape(na * nb, da + db)


def kernel(x, emb0, emb1, emb2, emb3, emb4, emb5, emb6, emb7, emb8):
    xt = x.T
    p12 = _pair(emb1, emb2)
    p34 = _pair(emb3, emb4)
    p56 = _pair(emb5, emb6)
    p78 = _pair(emb7, emb8)
    return _sc_call(xt, emb0, p12, p34, p56, p78)


# speedup vs baseline: 4.8393x; 4.2932x over previous
import functools

import jax
import jax.numpy as jnp
from jax import lax
from jax.experimental import pallas as pl
from jax.experimental.pallas import tpu as pltpu
from jax.experimental.pallas import tpu_sc as plsc

N = 100000
OUT_D = 128
BLK = 400
NBLK = N // BLK
NW = 32
ITERS = -(-NBLK // NW)

_T0 = 119 * 64
_PSIZES = (99 * 16, 108 * 16, 40 * 16, 4 * 16)
_POFF = []
_o = _T0
for _s in _PSIZES:
    _POFF.append(_o)
    _o += _s
_TAB_WORDS = _o

_PAIRS = ((1, 2, 11), (3, 4, 9), (5, 6, 8), (7, 8, 2))


def _sc_body(xt_h, tabs_h, out_h, tabs_v, idx_v, pi_v, row_v, wsem):
    pltpu.sync_copy(tabs_h, tabs_v)

    wid = lax.axis_index("s") * 2 + lax.axis_index("c")
    iota = lax.iota(jnp.int32, 16)
    ones = jnp.full((16,), 1, jnp.int32)
    lane128 = iota * 128

    def block_body(it, carry):
        blk = it * NW + wid

        @pl.when(blk < NBLK)
        def _():
            base = blk * BLK
            pltpu.sync_copy(xt_h.at[:, pl.ds(base, BLK)], idx_v)
            for t in range(BLK // 16):
                s = pl.ds(t * 16, 16)
                for k, (a, b, w) in enumerate(_PAIRS):
                    pi_v[k, s] = idx_v[a, s] * w + idx_v[b, s] + (_POFF[k] // 16)

            def group_body(g, c2):
                s = pl.ds(g * 16, 16)
                addr0 = idx_v[0, s] * 64
                sidx = lane128 + jnp.full((16,), 1, jnp.int32) * (g * 2048)
                addr = addr0
                for c in range(64):
                    val = plsc.load_gather(tabs_v, [addr])
                    plsc.store_scatter(row_v, [sidx], val)
                    addr = addr + ones
                    sidx = sidx + ones
                for k in range(4):
                    addr = pi_v[k, s] * 16
                    for c in range(16):
                        val = plsc.load_gather(tabs_v, [addr])
                        plsc.store_scatter(row_v, [sidx], val)
                        addr = addr + ones
                        sidx = sidx + ones
                return c2

            lax.fori_loop(0, BLK // 16, group_body, 0)
            pltpu.sync_copy(row_v, out_h.at[pl.ds(base * OUT_D, BLK * OUT_D)])

        return carry

    lax.fori_loop(0, ITERS, block_body, 0)


@functools.partial(jax.jit, static_argnums=())
def _sc_call(xt, tabs):
    mesh = plsc.VectorSubcoreMesh(core_axis_name="c", subcore_axis_name="s")
    fn = pl.kernel(
        _sc_body,
        out_type=jax.ShapeDtypeStruct((N * OUT_D,), jnp.float32),
        mesh=mesh,
        compiler_params=pltpu.CompilerParams(
            use_tc_tiling_on_sc=False, needs_layout_passes=False),
        scratch_types=[
            pltpu.VMEM((_TAB_WORDS,), jnp.float32),
            pltpu.VMEM((9, BLK), jnp.int32),
            pltpu.VMEM((4, BLK), jnp.int32),
            pltpu.VMEM((BLK * OUT_D,), jnp.float32),
            pltpu.SemaphoreType.DMA,
        ],
    )
    return fn(xt, tabs)


def _pair(a, b):
    na, da = a.shape
    nb, db = b.shape
    left = jnp.broadcast_to(a[:, None, :], (na, nb, da))
    right = jnp.broadcast_to(b[None, :, :], (na, nb, db))
    return jnp.concatenate([left, right], axis=-1).reshape(na * nb, da + db)


def kernel(x, emb0, emb1, emb2, emb3, emb4, emb5, emb6, emb7, emb8):
    xt = x.T
    tabs = jnp.concatenate([
        emb0.reshape(-1),
        _pair(emb1, emb2).reshape(-1),
        _pair(emb3, emb4).reshape(-1),
        _pair(emb5, emb6).reshape(-1),
        _pair(emb7, emb8).reshape(-1),
    ])
    return _sc_call(xt, tabs).reshape(N, OUT_D)


# speedup vs baseline: 6.8419x; 1.4138x over previous
import functools

import jax
import jax.numpy as jnp
from jax import lax
from jax.experimental import pallas as pl
from jax.experimental.pallas import tpu as pltpu
from jax.experimental.pallas import tpu_sc as plsc

N = 100000
OUT_D = 128
BLK = 400
NBLK = N // BLK
NW = 32
ITERS = -(-NBLK // NW)

_T0 = 119 * 64
_PSIZES = (99 * 16, 108 * 16, 40 * 16, 4 * 16)
_POFF = []
_o = _T0
for _s in _PSIZES:
    _POFF.append(_o)
    _o += _s
_TAB_WORDS = _o

_PAIRS = ((1, 2, 11), (3, 4, 9), (5, 6, 8), (7, 8, 2))


def _sc_body(xt_h, tabs_h, out_h, tabs_v, idx_v, pi_v, row_v, wsem):
    pltpu.sync_copy(tabs_h, tabs_v)

    wid = lax.axis_index("s") * 2 + lax.axis_index("c")
    iota = lax.iota(jnp.int32, 16)
    ones = jnp.full((16,), 1, jnp.int32)
    lane128 = iota * 128

    def block_body(it, carry):
        blk = it * NW + wid

        @pl.when(blk < NBLK)
        def _():
            base = blk * BLK
            pltpu.sync_copy(xt_h.at[:, pl.ds(base, BLK)], idx_v)
            for t in range(BLK // 16):
                s = pl.ds(t * 16, 16)
                for k, (a, b, w) in enumerate(_PAIRS):
                    pi_v[k, s] = idx_v[a, s] * w + idx_v[b, s] + (_POFF[k] // 16)

            def group_body(g, c2):
                s = pl.ds(g * 16, 16)
                addr0 = idx_v[0, s] * 64
                sidx0 = lane128 + jnp.full((16,), 1, jnp.int32) * (g * 2048)
                bases = [addr0 + 16 * j for j in range(4)]
                bases += [pi_v[k, s] * 16 for k in range(4)]
                for j, base_addr in enumerate(bases):
                    vals = [plsc.load_gather(tabs_v, [base_addr + c])
                            for c in range(16)]
                    sb = sidx0 + 16 * j
                    for c in range(16):
                        plsc.store_scatter(row_v, [sb + c], vals[c])
                return c2

            lax.fori_loop(0, BLK // 16, group_body, 0)
            pltpu.sync_copy(row_v, out_h.at[pl.ds(base * OUT_D, BLK * OUT_D)])

        return carry

    lax.fori_loop(0, ITERS, block_body, 0)


@functools.partial(jax.jit, static_argnums=())
def _sc_call(xt, tabs):
    mesh = plsc.VectorSubcoreMesh(core_axis_name="c", subcore_axis_name="s")
    fn = pl.kernel(
        _sc_body,
        out_type=jax.ShapeDtypeStruct((N * OUT_D,), jnp.float32),
        mesh=mesh,
        compiler_params=pltpu.CompilerParams(
            use_tc_tiling_on_sc=False, needs_layout_passes=False),
        scratch_types=[
            pltpu.VMEM((_TAB_WORDS,), jnp.float32),
            pltpu.VMEM((9, BLK), jnp.int32),
            pltpu.VMEM((4, BLK), jnp.int32),
            pltpu.VMEM((BLK * OUT_D,), jnp.float32),
            pltpu.SemaphoreType.DMA,
        ],
    )
    return fn(xt, tabs)


def _pair(a, b):
    na, da = a.shape
    nb, db = b.shape
    left = jnp.broadcast_to(a[:, None, :], (na, nb, da))
    right = jnp.broadcast_to(b[None, :, :], (na, nb, db))
    return jnp.concatenate([left, right], axis=-1).reshape(na * nb, da + db)


def kernel(x, emb0, emb1, emb2, emb3, emb4, emb5, emb6, emb7, emb8):
    xt = x.T
    tabs = jnp.concatenate([
        emb0.reshape(-1),
        _pair(emb1, emb2).reshape(-1),
        _pair(emb3, emb4).reshape(-1),
        _pair(emb5, emb6).reshape(-1),
        _pair(emb7, emb8).reshape(-1),
    ])
    return _sc_call(xt, tabs).reshape(N, OUT_D)


# speedup vs baseline: 7.0935x; 1.0368x over previous
import functools

import jax
import jax.numpy as jnp
from jax import lax
from jax.experimental import pallas as pl
from jax.experimental.pallas import tpu as pltpu
from jax.experimental.pallas import tpu_sc as plsc

N = 100000
OUT_D = 128
BLK = 400
NBLK = N // BLK
NW = 32
ITERS = -(-NBLK // NW)

_T0 = 119 * 64
_PSIZES = (99 * 16, 108 * 16, 40 * 16, 4 * 16)
_POFF = []
_o = _T0
for _s in _PSIZES:
    _POFF.append(_o)
    _o += _s
_TAB_WORDS = _o

_PAIRS = ((1, 2, 11), (3, 4, 9), (5, 6, 8), (7, 8, 2))


def _sc_body(xt_h, tabs_h, out_h,
             tabs_v, idxA, idxB, rowA, rowB,
             isemA, isemB, wsemA, wsemB):
    pltpu.sync_copy(tabs_h, tabs_v)

    wid = lax.axis_index("s") * 2 + lax.axis_index("c")
    iota = lax.iota(jnp.int32, 16)
    lane128 = iota * 128

    idx_bufs = (idxA, idxB)
    rows = (rowA, rowB)
    isems = (isemA, isemB)
    wsems = (wsemA, wsemB)

    def start_idx(it, cur):
        blk = it * NW + wid
        pltpu.async_copy(
            xt_h.at[:, pl.ds(blk * BLK, BLK)], idx_bufs[cur], isems[cur])

    def process(it, cur):
        blk = it * NW + wid
        idx_v, row_v = idx_bufs[cur], rows[cur]
        pltpu.make_async_copy(
            xt_h.at[:, pl.ds(0, BLK)], idx_v, isems[cur]).wait()
        if it >= 2:
            pltpu.make_async_copy(
                row_v, out_h.at[pl.ds(0, BLK * OUT_D)], wsems[cur]).wait()

        def group_body(g, c2):
            s = pl.ds(g * 16, 16)
            addr0 = idx_v[0, s] * 64
            sidx0 = lane128 + jnp.full((16,), 1, jnp.int32) * (g * 2048)
            bases = [addr0 + 16 * j for j in range(4)]
            bases += [(idx_v[a, s] * w + idx_v[b, s] + _POFF[k] // 16) * 16
                      for k, (a, b, w) in enumerate(_PAIRS)]
            for j, base_addr in enumerate(bases):
                vals = [plsc.load_gather(tabs_v, [base_addr + c])
                        for c in range(16)]
                sb = sidx0 + 16 * j
                for c in range(16):
                    plsc.store_scatter(row_v, [sb + c], vals[c])
            return c2

        lax.fori_loop(0, BLK // 16, group_body, 0)
        pltpu.async_copy(
            row_v, out_h.at[pl.ds(blk * BLK * OUT_D, BLK * OUT_D)],
            wsems[cur])

    start_idx(0, 0)
    for it in range(ITERS):
        cur = it & 1
        if it + 1 < ITERS:
            if it + 1 == ITERS - 1:
                @pl.when((it + 1) * NW + wid < NBLK)
                def _(it=it):
                    start_idx(it + 1, (it + 1) & 1)
            else:
                start_idx(it + 1, (it + 1) & 1)
        if it == ITERS - 1:
            @pl.when(it * NW + wid < NBLK)
            def _(it=it, cur=cur):
                process(it, cur)
        else:
            process(it, cur)
    pltpu.make_async_copy(
        rows[(ITERS - 2) & 1], out_h.at[pl.ds(0, BLK * OUT_D)],
        wsems[(ITERS - 2) & 1]).wait()

    @pl.when((ITERS - 1) * NW + wid < NBLK)
    def _():
        pltpu.make_async_copy(
            rows[(ITERS - 1) & 1], out_h.at[pl.ds(0, BLK * OUT_D)],
            wsems[(ITERS - 1) & 1]).wait()


@functools.partial(jax.jit, static_argnums=())
def _sc_call(xt, tabs):
    mesh = plsc.VectorSubcoreMesh(core_axis_name="c", subcore_axis_name="s")
    fn = pl.kernel(
        _sc_body,
        out_type=jax.ShapeDtypeStruct((N * OUT_D,), jnp.float32),
        mesh=mesh,
        compiler_params=pltpu.CompilerParams(
            use_tc_tiling_on_sc=False, needs_layout_passes=False),
        scratch_types=[
            pltpu.VMEM((_TAB_WORDS,), jnp.float32),
            pltpu.VMEM((9, BLK), jnp.int32),
            pltpu.VMEM((9, BLK), jnp.int32),
            pltpu.VMEM((BLK * OUT_D,), jnp.float32),
            pltpu.VMEM((BLK * OUT_D,), jnp.float32),
            pltpu.SemaphoreType.DMA,
            pltpu.SemaphoreType.DMA,
            pltpu.SemaphoreType.DMA,
            pltpu.SemaphoreType.DMA,
        ],
    )
    return fn(xt, tabs)


def _pair(a, b):
    na, da = a.shape
    nb, db = b.shape
    left = jnp.broadcast_to(a[:, None, :], (na, nb, da))
    right = jnp.broadcast_to(b[None, :, :], (na, nb, db))
    return jnp.concatenate([left, right], axis=-1).reshape(na * nb, da + db)


def kernel(x, emb0, emb1, emb2, emb3, emb4, emb5, emb6, emb7, emb8):
    xt = x.T
    tabs = jnp.concatenate([
        emb0.reshape(-1),
        _pair(emb1, emb2).reshape(-1),
        _pair(emb3, emb4).reshape(-1),
        _pair(emb5, emb6).reshape(-1),
        _pair(emb7, emb8).reshape(-1),
    ])
    return _sc_call(xt, tabs).reshape(N, OUT_D)


# speedup vs baseline: 24.5272x; 3.4577x over previous
import functools

import jax
import jax.numpy as jnp
from jax import lax
from jax.experimental import pallas as pl
from jax.experimental.pallas import tpu as pltpu
from jax.experimental.pallas import tpu_sc as plsc

N = 100000
OUT_D = 128
BLK = 400
NBLK = N // BLK
NW = 32
ITERS = -(-NBLK // NW)

_S0 = 65
_SP = 17
_T0 = 119 * _S0
_PSIZES = (99 * _SP, 108 * _SP, 40 * _SP, 4 * _SP)
_POFF = []
_o = _T0
for _s in _PSIZES:
    _POFF.append(_o)
    _o += _s
_TAB_WORDS = _o
_RS = OUT_D + 1

_PAIRS = ((1, 2, 11), (3, 4, 9), (5, 6, 8), (7, 8, 2))


def _sc_body(xt_h, tabs_h, out_h,
             tabs_v, idxA, idxB, rowA, rowB,
             isemA, isemB, wsemA, wsemB):
    pltpu.sync_copy(tabs_h, tabs_v)

    wid = lax.axis_index("s") * 2 + lax.axis_index("c")
    iota = lax.iota(jnp.int32, 16)
    lane128 = iota * 128

    idx_bufs = (idxA, idxB)
    rows = (rowA, rowB)
    isems = (isemA, isemB)
    wsems = (wsemA, wsemB)

    def start_idx(it, cur):
        blk = it * NW + wid
        pltpu.async_copy(
            xt_h.at[:, pl.ds(blk * BLK, BLK)], idx_bufs[cur], isems[cur])

    def process(it, cur):
        blk = it * NW + wid
        idx_v, row_v = idx_bufs[cur], rows[cur]
        pltpu.make_async_copy(
            xt_h.at[:, pl.ds(0, BLK)], idx_v, isems[cur]).wait()
        if it >= 2:
            pltpu.make_async_copy(
                row_v.at[:, pl.ds(0, OUT_D)],
                out_h.at[pl.ds(0, BLK), :], wsems[cur]).wait()

        ones = jnp.full((16,), 1, jnp.int32)

        def group_body(g, c2):
            s = pl.ds(g * 16, 16)
            addr0 = idx_v[0, s] * _S0
            ridx = iota + g * 16
            bases = [addr0 + 16 * j for j in range(4)]
            bases += [(idx_v[a, s] * w + idx_v[b, s]) * _SP + _POFF[k]
                      for k, (a, b, w) in enumerate(_PAIRS)]
            colv = jnp.full((16,), 0, jnp.int32)
            for j, base_addr in enumerate(bases):
                vals = [plsc.load_gather(tabs_v, [base_addr + c])
                        for c in range(16)]
                for c in range(16):
                    plsc.store_scatter(row_v, [ridx, colv], vals[c])
                    colv = colv + ones
            return c2

        lax.fori_loop(0, BLK // 16, group_body, 0)
        pltpu.async_copy(
            row_v.at[:, pl.ds(0, OUT_D)],
            out_h.at[pl.ds(blk * BLK, BLK), :], wsems[cur])

    start_idx(0, 0)
    for it in range(ITERS):
        cur = it & 1
        if it + 1 < ITERS:
            if it + 1 == ITERS - 1:
                @pl.when((it + 1) * NW + wid < NBLK)
                def _(it=it):
                    start_idx(it + 1, (it + 1) & 1)
            else:
                start_idx(it + 1, (it + 1) & 1)
        if it == ITERS - 1:
            @pl.when(it * NW + wid < NBLK)
            def _(it=it, cur=cur):
                process(it, cur)
        else:
            process(it, cur)
    pltpu.make_async_copy(
        rows[(ITERS - 2) & 1].at[:, pl.ds(0, OUT_D)],
        out_h.at[pl.ds(0, BLK), :], wsems[(ITERS - 2) & 1]).wait()

    @pl.when((ITERS - 1) * NW + wid < NBLK)
    def _():
        pltpu.make_async_copy(
            rows[(ITERS - 1) & 1].at[:, pl.ds(0, OUT_D)],
            out_h.at[pl.ds(0, BLK), :], wsems[(ITERS - 1) & 1]).wait()


@functools.partial(jax.jit, static_argnums=())
def _sc_call(xt, tabs):
    mesh = plsc.VectorSubcoreMesh(core_axis_name="c", subcore_axis_name="s")
    fn = pl.kernel(
        _sc_body,
        out_type=jax.ShapeDtypeStruct((N, OUT_D), jnp.float32),
        mesh=mesh,
        compiler_params=pltpu.CompilerParams(
            use_tc_tiling_on_sc=False, needs_layout_passes=False),
        scratch_types=[
            pltpu.VMEM((_TAB_WORDS,), jnp.float32),
            pltpu.VMEM((9, BLK), jnp.int32),
            pltpu.VMEM((9, BLK), jnp.int32),
            pltpu.VMEM((BLK, _RS), jnp.float32),
            pltpu.VMEM((BLK, _RS), jnp.float32),
            pltpu.SemaphoreType.DMA,
            pltpu.SemaphoreType.DMA,
            pltpu.SemaphoreType.DMA,
            pltpu.SemaphoreType.DMA,
        ],
    )
    return fn(xt, tabs)


def _pair(a, b):
    na, da = a.shape
    nb, db = b.shape
    left = jnp.broadcast_to(a[:, None, :], (na, nb, da))
    right = jnp.broadcast_to(b[None, :, :], (na, nb, db))
    return jnp.concatenate([left, right], axis=-1).reshape(na * nb, da + db)


def _padrow(t):
    return jnp.pad(t, ((0, 0), (0, 1))).reshape(-1)


def kernel(x, emb0, emb1, emb2, emb3, emb4, emb5, emb6, emb7, emb8):
    xt = x.T
    tabs = jnp.concatenate([
        _padrow(emb0),
        _padrow(_pair(emb1, emb2)),
        _padrow(_pair(emb3, emb4)),
        _padrow(_pair(emb5, emb6)),
        _padrow(_pair(emb7, emb8)),
    ])
    return _sc_call(xt, tabs)


# speedup vs baseline: 33.9594x; 1.3846x over previous
import functools

import jax
import jax.numpy as jnp
from jax import lax
from jax.experimental import pallas as pl
from jax.experimental.pallas import tpu as pltpu
from jax.experimental.pallas import tpu_sc as plsc

N = 100000
OUT_D = 128
BLK = 400
NBLK = N // BLK
NW = 32
ITERS = -(-NBLK // NW)

_T0 = 119 * 64
_PSIZES = (99 * 16, 108 * 16, 40 * 16, 4 * 16)
_POFF = []
_o = _T0
for _s in _PSIZES:
    _POFF.append(_o)
    _o += _s
_TAB_WORDS = _o

_PAIRS = ((1, 2, 11), (3, 4, 9), (5, 6, 8), (7, 8, 2))

def _sc_body(xt_h, tabs_h, out_h,
             tabs_v, idxA, idxB, rowA, rowB,
             isemA, isemB, wsemA, wsemB):
    pltpu.sync_copy(tabs_h, tabs_v)

    wid = lax.axis_index("s") * 2 + lax.axis_index("c")
    iota = lax.iota(jnp.int32, 16)

    idx_bufs = (idxA, idxB)
    rows = (rowA, rowB)
    isems = (isemA, isemB)
    wsems = (wsemA, wsemB)

    def start_idx(it, cur):
        blk = it * NW + wid
        pltpu.async_copy(
            xt_h.at[:, pl.ds(blk * BLK, BLK)], idx_bufs[cur], isems[cur])

    def process(it, cur):
        blk = it * NW + wid
        idx_v, row_v = idx_bufs[cur], rows[cur]
        pltpu.make_async_copy(
            xt_h.at[:, pl.ds(0, BLK)], idx_v, isems[cur]).wait()
        if it >= 2:
            pltpu.make_async_copy(
                row_v, out_h.at[pl.ds(0, BLK), :], wsems[cur]).wait()

        def group_body(g, c2):
            s = pl.ds(g * 16, 16)
            base0 = idx_v[0, s] * 64
            pbases = [(idx_v[a, s] * w + idx_v[b, s]) * 16 + _POFF[k]
                      for k, (a, b, w) in enumerate(_PAIRS)]
            iotas = [iota + 16 * j for j in range(4)]
            for r in range(16):
                rowi = g * 16 + r
                ridx = jnp.full((16,), r, jnp.int32)
                b0 = base0.at[ridx].get(mode="promise_in_bounds")
                vals = [plsc.load_gather(tabs_v, [b0 + iotas[j]])
                        for j in range(4)]
                for k in range(4):
                    bk = pbases[k].at[ridx].get(mode="promise_in_bounds")
                    vals.append(plsc.load_gather(tabs_v, [bk + iota]))
                for j in range(8):
                    row_v[rowi, pl.ds(16 * j, 16)] = vals[j]
            return c2

        lax.fori_loop(0, BLK // 16, group_body, 0)
        pltpu.async_copy(row_v, out_h.at[pl.ds(blk * BLK, BLK), :],
                         wsems[cur])

    start_idx(0, 0)
    for it in range(ITERS):
        cur = it & 1
        if it + 1 < ITERS:
            if it + 1 == ITERS - 1:
                @pl.when((it + 1) * NW + wid < NBLK)
                def _(it=it):
                    start_idx(it + 1, (it + 1) & 1)
            else:
                start_idx(it + 1, (it + 1) & 1)
        if it == ITERS - 1:
            @pl.when(it * NW + wid < NBLK)
            def _(it=it, cur=cur):
                process(it, cur)
        else:
            process(it, cur)
    pltpu.make_async_copy(
        rows[(ITERS - 2) & 1], out_h.at[pl.ds(0, BLK), :],
        wsems[(ITERS - 2) & 1]).wait()

    @pl.when((ITERS - 1) * NW + wid < NBLK)
    def _():
        pltpu.make_async_copy(
            rows[(ITERS - 1) & 1], out_h.at[pl.ds(0, BLK), :],
            wsems[(ITERS - 1) & 1]).wait()


@functools.partial(jax.jit, static_argnums=())
def _sc_call(xt, tabs):
    mesh = plsc.VectorSubcoreMesh(core_axis_name="c", subcore_axis_name="s")
    fn = pl.kernel(
        _sc_body,
        out_type=jax.ShapeDtypeStruct((N, OUT_D), jnp.float32),
        mesh=mesh,
        compiler_params=pltpu.CompilerParams(
            use_tc_tiling_on_sc=False, needs_layout_passes=False),
        scratch_types=[
            pltpu.VMEM((_TAB_WORDS,), jnp.float32),
            pltpu.VMEM((9, BLK), jnp.int32),
            pltpu.VMEM((9, BLK), jnp.int32),
            pltpu.VMEM((BLK, OUT_D), jnp.float32),
            pltpu.VMEM((BLK, OUT_D), jnp.float32),
            pltpu.SemaphoreType.DMA,
            pltpu.SemaphoreType.DMA,
            pltpu.SemaphoreType.DMA,
            pltpu.SemaphoreType.DMA,
        ],
    )
    return fn(xt, tabs)


def _pair(a, b):
    na, da = a.shape
    nb, db = b.shape
    left = jnp.broadcast_to(a[:, None, :], (na, nb, da))
    right = jnp.broadcast_to(b[None, :, :], (na, nb, db))
    return jnp.concatenate([left, right], axis=-1).reshape(na * nb, da + db)


def kernel(x, emb0, emb1, emb2, emb3, emb4, emb5, emb6, emb7, emb8):
    xt = x.T
    tabs = jnp.concatenate([
        emb0.reshape(-1),
        _pair(emb1, emb2).reshape(-1),
        _pair(emb3, emb4).reshape(-1),
        _pair(emb5, emb6).reshape(-1),
        _pair(emb7, emb8).reshape(-1),
    ])
    return _sc_call(xt, tabs)


# speedup vs baseline: 37.1775x; 1.0948x over previous
import functools

import jax
import jax.numpy as jnp
from jax import lax
from jax.experimental import pallas as pl
from jax.experimental.pallas import tpu as pltpu
from jax.experimental.pallas import tpu_sc as plsc

N = 100000
OUT_D = 128
BLK = 400
NBLK = N // BLK
NW = 32
ITERS = -(-NBLK // NW)

_T0 = 119 * 64
_PSIZES = (99 * 16, 108 * 16, 160 * 32)
_POFF = []
_o = _T0
for _s in _PSIZES:
    _POFF.append(_o)
    _o += _s
_TAB_WORDS = _o

_PAIRS = ((1, 2, 11), (3, 4, 9))

def _sc_body(xt_h, tabs_h, out_h,
             tabs_v, idxA, idxB, rowA, rowB,
             isemA, isemB, wsemA, wsemB):
    pltpu.sync_copy(tabs_h, tabs_v)

    wid = lax.axis_index("s") * 2 + lax.axis_index("c")
    iota = lax.iota(jnp.int32, 16)

    idx_bufs = (idxA, idxB)
    rows = (rowA, rowB)
    isems = (isemA, isemB)
    wsems = (wsemA, wsemB)

    def start_idx(it, cur):
        blk = it * NW + wid
        pltpu.async_copy(
            xt_h.at[:, pl.ds(blk * BLK, BLK)], idx_bufs[cur], isems[cur])

    def process(it, cur):
        blk = it * NW + wid
        idx_v, row_v = idx_bufs[cur], rows[cur]
        pltpu.make_async_copy(
            xt_h.at[:, pl.ds(0, BLK)], idx_v, isems[cur]).wait()
        if it >= 2:
            pltpu.make_async_copy(
                row_v, out_h.at[pl.ds(0, BLK), :], wsems[cur]).wait()

        @plsc.parallel_loop(0, BLK // 16, 1, unroll=2)
        def group_body(g):
            s = pl.ds(g * 16, 16)
            base0 = idx_v[0, s] * 64
            pbases = [(idx_v[a, s] * w + idx_v[b, s]) * 16 + _POFF[k]
                      for k, (a, b, w) in enumerate(_PAIRS)]
            q = idx_v[5, s]
            q = q * 8 + idx_v[6, s]
            q = q * 2 + idx_v[7, s]
            q = q * 2 + idx_v[8, s]
            pbases.append(q * 32 + _POFF[2])
            iotas = [iota + 16 * j for j in range(4)]
            for r in range(16):
                rowi = g * 16 + r
                ridx = jnp.full((16,), r, jnp.int32)
                b0 = base0.at[ridx].get(mode="promise_in_bounds")
                vals = [plsc.load_gather(tabs_v, [b0 + iotas[j]])
                        for j in range(4)]
                for k in range(2):
                    bk = pbases[k].at[ridx].get(mode="promise_in_bounds")
                    vals.append(plsc.load_gather(tabs_v, [bk + iota]))
                bq = pbases[2].at[ridx].get(mode="promise_in_bounds")
                vals.append(plsc.load_gather(tabs_v, [bq + iotas[0]]))
                vals.append(plsc.load_gather(tabs_v, [bq + iotas[1]]))
                for j in range(8):
                    row_v[rowi, pl.ds(16 * j, 16)] = vals[j]
        pltpu.async_copy(row_v, out_h.at[pl.ds(blk * BLK, BLK), :],
                         wsems[cur])

    start_idx(0, 0)
    for it in range(ITERS):
        cur = it & 1
        if it + 1 < ITERS:
            if it + 1 == ITERS - 1:
                @pl.when((it + 1) * NW + wid < NBLK)
                def _(it=it):
                    start_idx(it + 1, (it + 1) & 1)
            else:
                start_idx(it + 1, (it + 1) & 1)
        if it == ITERS - 1:
            @pl.when(it * NW + wid < NBLK)
            def _(it=it, cur=cur):
                process(it, cur)
        else:
            process(it, cur)
    pltpu.make_async_copy(
        rows[(ITERS - 2) & 1], out_h.at[pl.ds(0, BLK), :],
        wsems[(ITERS - 2) & 1]).wait()

    @pl.when((ITERS - 1) * NW + wid < NBLK)
    def _():
        pltpu.make_async_copy(
            rows[(ITERS - 1) & 1], out_h.at[pl.ds(0, BLK), :],
            wsems[(ITERS - 1) & 1]).wait()


@functools.partial(jax.jit, static_argnums=())
def _sc_call(xt, tabs):
    mesh = plsc.VectorSubcoreMesh(core_axis_name="c", subcore_axis_name="s")
    fn = pl.kernel(
        _sc_body,
        out_type=jax.ShapeDtypeStruct((N, OUT_D), jnp.float32),
        mesh=mesh,
        compiler_params=pltpu.CompilerParams(
            use_tc_tiling_on_sc=False, needs_layout_passes=False),
        scratch_types=[
            pltpu.VMEM((_TAB_WORDS,), jnp.float32),
            pltpu.VMEM((9, BLK), jnp.int32),
            pltpu.VMEM((9, BLK), jnp.int32),
            pltpu.VMEM((BLK, OUT_D), jnp.float32),
            pltpu.VMEM((BLK, OUT_D), jnp.float32),
            pltpu.SemaphoreType.DMA,
            pltpu.SemaphoreType.DMA,
            pltpu.SemaphoreType.DMA,
            pltpu.SemaphoreType.DMA,
        ],
    )
    return fn(xt, tabs)


def _pair(a, b):
    na, da = a.shape
    nb, db = b.shape
    left = jnp.broadcast_to(a[:, None, :], (na, nb, da))
    right = jnp.broadcast_to(b[None, :, :], (na, nb, db))
    return jnp.concatenate([left, right], axis=-1).reshape(na * nb, da + db)


def kernel(x, emb0, emb1, emb2, emb3, emb4, emb5, emb6, emb7, emb8):
    xt = x.T
    q5678 = _pair(_pair(emb5, emb6), _pair(emb7, emb8))
    tabs = jnp.concatenate([
        emb0.reshape(-1),
        _pair(emb1, emb2).reshape(-1),
        _pair(emb3, emb4).reshape(-1),
        q5678.reshape(-1),
    ])
    return _sc_call(xt, tabs)
